# baseline (speedup 1.0000x reference)
import functools

import numpy as np

import jax
import jax.numpy as jnp
from jax import lax
from jax.experimental import pallas as pl
from jax.experimental.pallas import tpu as pltpu
from jax.experimental.pallas import tpu_sc as plsc

N_NODES = 10000
IN_CH = 128
HID_CH = 128
OUT_CH = 64
N_EDGES = 320000

NC, NS, LANES = 2, 16, 16
NW = NC * NS

EPT_RAW = N_EDGES // NW
CHUNK = 128
NCHUNK = 80
EPT = NCHUNK * CHUNK
NPAD = 10112
RPT = NPAD // NS

_mesh = plsc.VectorSubcoreMesh(core_axis_name="c", subcore_axis_name="s")


def _deg_body(dst_hbm, out_hbm, idx_v, hist_v):
    c = lax.axis_index("c")
    s = lax.axis_index("s")
    wid = c * NS + s
    pltpu.sync_copy(dst_hbm.at[wid], idx_v)
    zero16 = jnp.zeros((LANES,), jnp.float32)

    def zrow(r, carry):
        for cc in range(CHUNK // LANES):
            hist_v[r, pl.ds(cc * LANES, LANES)] = zero16
        return carry

    lax.fori_loop(0, NCHUNK, zrow, 0)

    ones = jnp.ones((LANES,), jnp.float32)

    def edge(j, carry):
        idx = idx_v[pl.ds(j * LANES, LANES)]
        hi = lax.shift_right_logical(idx, 7)
        lo = lax.bitwise_and(idx, 127)
        plsc.addupdate_scatter(hist_v, [hi, lo], ones)
        return carry

    lax.fori_loop(0, EPT_RAW // LANES, edge, 0)
    pltpu.sync_copy(hist_v, out_hbm.at[wid])


_deg_call = pl.kernel(
    _deg_body,
    out_type=jax.ShapeDtypeStruct((NW, NCHUNK, CHUNK), jnp.float32),
    mesh=_mesh,
    scratch_types=[
        pltpu.VMEM((EPT_RAW,), jnp.int32),
        pltpu.VMEM((NCHUNK, CHUNK), jnp.float32),
    ],
    compiler_params=pltpu.CompilerParams(needs_layout_passes=False),
)


NF = 150
NSL = 8
ACH = 128
TOTCH = 16 * (NF + NSL)
STAGE = 17 * NF + 15 * NSL


def _make_agg(d, depth):
    def body(hs_hbm, src_hbm, dst_hbm, zeros_hbm, out_hbm,
             src_v, dst_v, rows_v, acc_sh, sem_g, sem_s):
        c = lax.axis_index("c")
        s = lax.axis_index("s")
        nch = jnp.where(c == 0, NF, NSL)
        off = jnp.where(c == 0, s * NF, 16 * NF + s * NSL)
        pltpu.sync_copy(src_hbm.at[pl.ds(off, NF)], src_v)
        pltpu.sync_copy(dst_hbm.at[pl.ds(off, NF)], dst_v)
        pltpu.sync_copy(zeros_hbm, acc_sh.at[pl.ds(s * RPT, RPT)])
        plsc.subcore_barrier()

        for k in range(depth):
            pltpu.async_copy(hs_hbm.at[src_v.at[k]], rows_v.at[k], sem_g)

        def step(j, carry):
            buf = lax.rem(j, depth)
            pltpu.make_async_copy(hs_hbm.at[src_v.at[j]], rows_v.at[buf],
                                  sem_g).wait()
            pltpu.async_copy(rows_v.at[buf], acc_sh.at[dst_v.at[j]], sem_s,
                             add=True)

            @pl.when(jnp.logical_and(j >= 1, j - 1 + depth < nch))
            def _():
                pbuf = lax.rem(j - 1, depth)
                pltpu.make_async_copy(rows_v.at[pbuf],
                                      acc_sh.at[dst_v.at[j - 1]], sem_s).wait()
                pltpu.async_copy(hs_hbm.at[src_v.at[j - 1 + depth]],
                                 rows_v.at[pbuf], sem_g)

            return carry

        lax.fori_loop(0, nch, step, 0)
        for k in range(depth):
            pltpu.make_async_copy(rows_v.at[k],
                                  acc_sh.at[dst_v.at[0]], sem_s).wait()
        plsc.subcore_barrier()
        pltpu.sync_copy(acc_sh.at[pl.ds(s * RPT, RPT)],
                        out_hbm.at[c].at[pl.ds(s * RPT, RPT)])

    return pl.kernel(
        body,
        out_type=jax.ShapeDtypeStruct((NC, NPAD, d), jnp.float32),
        mesh=_mesh,
        scratch_types=[
            pltpu.VMEM((NF, ACH), jnp.int32),
            pltpu.VMEM((NF, ACH), jnp.int32),
            pltpu.VMEM((depth, ACH, d), jnp.float32),
            pltpu.VMEM_SHARED((NPAD, d), jnp.float32),
            pltpu.SemaphoreType.DMA,
            pltpu.SemaphoreType.DMA,
        ],
        compiler_params=pltpu.CompilerParams(use_tc_tiling_on_sc=False),
    )


_agg64 = _make_agg(OUT_CH, 6)


def _dinv_body(degp_ref, out_ref):
    deg = jnp.sum(degp_ref[...], axis=0) + 1.0
    out_ref[...] = lax.rsqrt(deg)


def _hs1_body(x_ref, w_ref, dinv_ref, outa_ref, outb_ref):
    y = jnp.dot(x_ref[...], w_ref[...], preferred_element_type=jnp.float32)
    y = y * dinv_ref[...]
    outa_ref[...] = y[:, :OUT_CH]
    outb_ref[...] = y[:, OUT_CH:]


def _hs2_body(aggA0, aggA1, aggB0, aggB1, hs1a_ref, hs1b_ref, dinv_ref,
              b1_ref, w2_ref, out_ref):
    dinv = dinv_ref[...]
    ha = (aggA0[0] + aggA1[0] + hs1a_ref[...]) * dinv + b1_ref[:, :OUT_CH]
    hb = (aggB0[0] + aggB1[0] + hs1b_ref[...]) * dinv + b1_ref[:, OUT_CH:]
    ha = jnp.maximum(ha, 0.0)
    hb = jnp.maximum(hb, 0.0)
    out_ref[...] = (
        jnp.dot(ha, w2_ref[:OUT_CH], preferred_element_type=jnp.float32)
        + jnp.dot(hb, w2_ref[OUT_CH:], preferred_element_type=jnp.float32)
    ) * dinv


def _z_body(aggp_ref0, aggp_ref1, hs2_ref, dinv_ref, b2_ref, out_ref):
    out_ref[...] = (
        (aggp_ref0[0] + aggp_ref1[0] + hs2_ref[...]) * dinv_ref[...]
        + b2_ref[...]
    )


def _adj_body(zr_ref, zc_ref, out_ref):
    g = lax.dot_general(zr_ref[...], zc_ref[...],
                        (((1,), (1,)), ((), ())),
                        preferred_element_type=jnp.float32)
    out_ref[...] = 0.5 * jnp.tanh(0.5 * g) + 0.5


_BM = 512
_GM = -(-N_NODES // _BM)

_dinv_call = pl.pallas_call(
    _dinv_body,
    out_shape=jax.ShapeDtypeStruct((NCHUNK, CHUNK), jnp.float32),
    in_specs=[pl.BlockSpec((NW, NCHUNK, CHUNK), lambda: (0, 0, 0))],
    out_specs=pl.BlockSpec((NCHUNK, CHUNK), lambda: (0, 0)),
)

_hs1_call = pl.pallas_call(
    _hs1_body,
    grid=(_GM,),
    out_shape=[jax.ShapeDtypeStruct((N_NODES, OUT_CH), jnp.float32),
               jax.ShapeDtypeStruct((N_NODES, OUT_CH), jnp.float32)],
    in_specs=[
        pl.BlockSpec((_BM, IN_CH), lambda i: (i, 0)),
        pl.BlockSpec((IN_CH, HID_CH), lambda i: (0, 0)),
        pl.BlockSpec((_BM, 1), lambda i: (i, 0)),
    ],
    out_specs=[pl.BlockSpec((_BM, OUT_CH), lambda i: (i, 0)),
               pl.BlockSpec((_BM, OUT_CH), lambda i: (i, 0))],
)

_hs2_call = pl.pallas_call(
    _hs2_body,
    grid=(_GM,),
    out_shape=jax.ShapeDtypeStruct((N_NODES, OUT_CH), jnp.float32),
    in_specs=[
        pl.BlockSpec((1, _BM, OUT_CH), lambda i: (0, i, 0)),
        pl.BlockSpec((1, _BM, OUT_CH), lambda i: (1, i, 0)),
        pl.BlockSpec((1, _BM, OUT_CH), lambda i: (0, i, 0)),
        pl.BlockSpec((1, _BM, OUT_CH), lambda i: (1, i, 0)),
        pl.BlockSpec((_BM, OUT_CH), lambda i: (i, 0)),
        pl.BlockSpec((_BM, OUT_CH), lambda i: (i, 0)),
        pl.BlockSpec((_BM, 1), lambda i: (i, 0)),
        pl.BlockSpec((1, HID_CH), lambda i: (0, 0)),
        pl.BlockSpec((HID_CH, OUT_CH), lambda i: (0, 0)),
    ],
    out_specs=pl.BlockSpec((_BM, OUT_CH), lambda i: (i, 0)),
)

_z_call = pl.pallas_call(
    _z_body,
    grid=(_GM,),
    out_shape=jax.ShapeDtypeStruct((N_NODES, OUT_CH), jnp.float32),
    in_specs=[
        pl.BlockSpec((1, _BM, OUT_CH), lambda i: (0, i, 0)),
        pl.BlockSpec((1, _BM, OUT_CH), lambda i: (1, i, 0)),
        pl.BlockSpec((_BM, OUT_CH), lambda i: (i, 0)),
        pl.BlockSpec((_BM, 1), lambda i: (i, 0)),
        pl.BlockSpec((1, OUT_CH), lambda i: (0, 0)),
    ],
    out_specs=pl.BlockSpec((_BM, OUT_CH), lambda i: (i, 0)),
)

_BN = 4096
_GN = -(-N_NODES // _BN)

_adj_call = pl.pallas_call(
    _adj_body,
    grid=(_GM, _GN),
    out_shape=jax.ShapeDtypeStruct((N_NODES, N_NODES), jnp.float32),
    in_specs=[
        pl.BlockSpec((_BM, OUT_CH), lambda i, j: (i, 0)),
        pl.BlockSpec((_BN, OUT_CH), lambda i, j: (j, 0)),
    ],
    out_specs=pl.BlockSpec((_BM, _BN), lambda i, j: (i, j)),
    compiler_params=pltpu.CompilerParams(
        dimension_semantics=("parallel", "parallel")),
)


_PADN = STAGE * ACH - N_EDGES
_TRASH = jnp.asarray(N_NODES + np.arange(_PADN) % (NPAD - N_NODES), jnp.int32)
_SRCPAD = jnp.zeros((_PADN,), jnp.int32)
_ZEROS64 = jnp.zeros((RPT, OUT_CH), jnp.float32)


def kernel(x, edge_index, W1, b1, W2, b2):
    ei = edge_index.astype(jnp.int32)
    src, dst = ei[0], ei[1]

    degp = _deg_call(dst.reshape(NW, EPT_RAW))
    dinv2d = _dinv_call(degp)
    dinv = dinv2d.reshape(-1)[:N_NODES, None]

    dstp = jnp.concatenate([dst, _TRASH]).reshape(STAGE, ACH)
    srcp = jnp.concatenate([src, _SRCPAD]).reshape(STAGE, ACH)

    zeros64 = _ZEROS64
    hs1a, hs1b = _hs1_call(x, W1, dinv)
    aggp1a = _agg64(hs1a, srcp, dstp, zeros64)
    aggp1b = _agg64(hs1b, srcp, dstp, zeros64)
    hs2 = _hs2_call(aggp1a, aggp1a, aggp1b, aggp1b, hs1a, hs1b, dinv,
                    b1.reshape(1, HID_CH), W2)
    aggp2 = _agg64(hs2, srcp, dstp, zeros64)
    z = _z_call(aggp2, aggp2, hs2, dinv, b2.reshape(1, OUT_CH))
    adj = _adj_call(z, z)
    return (z, adj)


# speedup vs baseline: 1.0061x; 1.0061x over previous
import functools

import jax
import jax.numpy as jnp
from jax import lax
from jax.experimental import pallas as pl
from jax.experimental.pallas import tpu as pltpu
from jax.experimental.pallas import tpu_sc as plsc

N_NODES = 10000
IN_CH = 128
HID_CH = 128
OUT_CH = 64
N_EDGES = 320000

NC, NS, LANES = 2, 16, 16
NW = NC * NS

EPT_RAW = N_EDGES // NW
CHUNK = 128
NCHUNK = 80
EPT = NCHUNK * CHUNK
NPAD = 10112
RPT = NPAD // NS

_mesh = plsc.VectorSubcoreMesh(core_axis_name="c", subcore_axis_name="s")


def _deg_body(dst_hbm, out_hbm, idx_v, hist_v):
    c = lax.axis_index("c")
    s = lax.axis_index("s")
    wid = c * NS + s
    pltpu.sync_copy(dst_hbm.at[wid], idx_v)
    zero16 = jnp.zeros((LANES,), jnp.float32)

    def zrow(r, carry):
        for cc in range(CHUNK // LANES):
            hist_v[r, pl.ds(cc * LANES, LANES)] = zero16
        return carry

    lax.fori_loop(0, NCHUNK, zrow, 0)

    ones = jnp.ones((LANES,), jnp.float32)

    def edge(j, carry):
        idx = idx_v[pl.ds(j * LANES, LANES)]
        hi = lax.shift_right_logical(idx, 7)
        lo = lax.bitwise_and(idx, 127)
        plsc.addupdate_scatter(hist_v, [hi, lo], ones)
        return carry

    lax.fori_loop(0, EPT_RAW // LANES, edge, 0)
    pltpu.sync_copy(hist_v, out_hbm.at[wid])


_deg_call = pl.kernel(
    _deg_body,
    out_type=jax.ShapeDtypeStruct((NW, NCHUNK, CHUNK), jnp.float32),
    mesh=_mesh,
    scratch_types=[
        pltpu.VMEM((EPT_RAW,), jnp.int32),
        pltpu.VMEM((NCHUNK, CHUNK), jnp.float32),
    ],
    compiler_params=pltpu.CompilerParams(needs_layout_passes=False),
)


NF = 150
NSL = 8
ACH = 128
TOTCH = 16 * (NF + NSL)
STAGE = 17 * NF + 15 * NSL


def _make_agg(d, depth):
    def body(hs_hbm, src_hbm, dst_hbm, zeros_hbm, out_hbm,
             src_v, dst_v, rows_v, acc_sh, sem_g, sem_s):
        c = lax.axis_index("c")
        s = lax.axis_index("s")
        nch = jnp.where(c == 0, NF, NSL)
        off = jnp.where(c == 0, s * NF, 16 * NF + s * NSL)
        pltpu.sync_copy(src_hbm.at[pl.ds(off, NF)], src_v)
        pltpu.sync_copy(dst_hbm.at[pl.ds(off, NF)], dst_v)
        pltpu.sync_copy(zeros_hbm, acc_sh.at[pl.ds(s * RPT, RPT)])
        plsc.subcore_barrier()

        for k in range(depth):
            pltpu.async_copy(hs_hbm.at[src_v.at[k]], rows_v.at[k], sem_g)

        def step(j, carry):
            buf = lax.rem(j, depth)
            pltpu.make_async_copy(hs_hbm.at[src_v.at[j]], rows_v.at[buf],
                                  sem_g).wait()
            pltpu.async_copy(rows_v.at[buf], acc_sh.at[dst_v.at[j]], sem_s,
                             add=True)

            @pl.when(jnp.logical_and(j >= 1, j - 1 + depth < nch))
            def _():
                pbuf = lax.rem(j - 1, depth)
                pltpu.make_async_copy(rows_v.at[pbuf],
                                      acc_sh.at[dst_v.at[j - 1]], sem_s).wait()
                pltpu.async_copy(hs_hbm.at[src_v.at[j - 1 + depth]],
                                 rows_v.at[pbuf], sem_g)

            return carry

        lax.fori_loop(0, nch, step, 0)
        for k in range(depth):
            pltpu.make_async_copy(rows_v.at[k],
                                  acc_sh.at[dst_v.at[0]], sem_s).wait()
        plsc.subcore_barrier()
        pltpu.sync_copy(acc_sh.at[pl.ds(s * RPT, RPT)],
                        out_hbm.at[c].at[pl.ds(s * RPT, RPT)])

    return pl.kernel(
        body,
        out_type=jax.ShapeDtypeStruct((NC, NPAD, d), jnp.float32),
        mesh=_mesh,
        scratch_types=[
            pltpu.VMEM((NF, ACH), jnp.int32),
            pltpu.VMEM((NF, ACH), jnp.int32),
            pltpu.VMEM((depth, ACH, d), jnp.float32),
            pltpu.VMEM_SHARED((NPAD, d), jnp.float32),
            pltpu.SemaphoreType.DMA,
            pltpu.SemaphoreType.DMA,
        ],
        compiler_params=pltpu.CompilerParams(use_tc_tiling_on_sc=False),
    )


_agg64 = _make_agg(OUT_CH, 6)


def _dinv_body(degp_ref, out_ref):
    deg = jnp.sum(degp_ref[...], axis=0) + 1.0
    out_ref[...] = lax.rsqrt(deg)


def _hs1_body(x_ref, w_ref, dinv_ref, outa_ref, outb_ref):
    y = jnp.dot(x_ref[...], w_ref[...], preferred_element_type=jnp.float32)
    y = y * dinv_ref[...]
    outa_ref[...] = y[:, :OUT_CH]
    outb_ref[...] = y[:, OUT_CH:]


def _hs2_body(aggA0, aggA1, aggB0, aggB1, hs1a_ref, hs1b_ref, dinv_ref,
              b1_ref, w2_ref, out_ref):
    dinv = dinv_ref[...]
    ha = (aggA0[0] + aggA1[0] + hs1a_ref[...]) * dinv + b1_ref[:, :OUT_CH]
    hb = (aggB0[0] + aggB1[0] + hs1b_ref[...]) * dinv + b1_ref[:, OUT_CH:]
    ha = jnp.maximum(ha, 0.0)
    hb = jnp.maximum(hb, 0.0)
    out_ref[...] = (
        jnp.dot(ha, w2_ref[:OUT_CH], preferred_element_type=jnp.float32)
        + jnp.dot(hb, w2_ref[OUT_CH:], preferred_element_type=jnp.float32)
    ) * dinv


def _z_body(aggp_ref0, aggp_ref1, hs2_ref, dinv_ref, b2_ref, out_ref):
    out_ref[...] = (
        (aggp_ref0[0] + aggp_ref1[0] + hs2_ref[...]) * dinv_ref[...]
        + b2_ref[...]
    )


def _adj_body(zr_ref, zc_ref, out_ref):
    g = lax.dot_general(zr_ref[...], zc_ref[...],
                        (((1,), (1,)), ((), ())),
                        preferred_element_type=jnp.float32)
    out_ref[...] = 0.5 * jnp.tanh(0.5 * g) + 0.5


_BM = 512
_GM = -(-N_NODES // _BM)

_dinv_call = pl.pallas_call(
    _dinv_body,
    out_shape=jax.ShapeDtypeStruct((NCHUNK, CHUNK), jnp.float32),
    in_specs=[pl.BlockSpec((NW, NCHUNK, CHUNK), lambda: (0, 0, 0))],
    out_specs=pl.BlockSpec((NCHUNK, CHUNK), lambda: (0, 0)),
)

_hs1_call = pl.pallas_call(
    _hs1_body,
    grid=(_GM,),
    out_shape=[jax.ShapeDtypeStruct((N_NODES, OUT_CH), jnp.float32),
               jax.ShapeDtypeStruct((N_NODES, OUT_CH), jnp.float32)],
    in_specs=[
        pl.BlockSpec((_BM, IN_CH), lambda i: (i, 0)),
        pl.BlockSpec((IN_CH, HID_CH), lambda i: (0, 0)),
        pl.BlockSpec((_BM, 1), lambda i: (i, 0)),
    ],
    out_specs=[pl.BlockSpec((_BM, OUT_CH), lambda i: (i, 0)),
               pl.BlockSpec((_BM, OUT_CH), lambda i: (i, 0))],
)

_hs2_call = pl.pallas_call(
    _hs2_body,
    grid=(_GM,),
    out_shape=jax.ShapeDtypeStruct((N_NODES, OUT_CH), jnp.float32),
    in_specs=[
        pl.BlockSpec((1, _BM, OUT_CH), lambda i: (0, i, 0)),
        pl.BlockSpec((1, _BM, OUT_CH), lambda i: (1, i, 0)),
        pl.BlockSpec((1, _BM, OUT_CH), lambda i: (0, i, 0)),
        pl.BlockSpec((1, _BM, OUT_CH), lambda i: (1, i, 0)),
        pl.BlockSpec((_BM, OUT_CH), lambda i: (i, 0)),
        pl.BlockSpec((_BM, OUT_CH), lambda i: (i, 0)),
        pl.BlockSpec((_BM, 1), lambda i: (i, 0)),
        pl.BlockSpec((1, HID_CH), lambda i: (0, 0)),
        pl.BlockSpec((HID_CH, OUT_CH), lambda i: (0, 0)),
    ],
    out_specs=pl.BlockSpec((_BM, OUT_CH), lambda i: (i, 0)),
)

_z_call = pl.pallas_call(
    _z_body,
    grid=(_GM,),
    out_shape=jax.ShapeDtypeStruct((N_NODES, OUT_CH), jnp.float32),
    in_specs=[
        pl.BlockSpec((1, _BM, OUT_CH), lambda i: (0, i, 0)),
        pl.BlockSpec((1, _BM, OUT_CH), lambda i: (1, i, 0)),
        pl.BlockSpec((_BM, OUT_CH), lambda i: (i, 0)),
        pl.BlockSpec((_BM, 1), lambda i: (i, 0)),
        pl.BlockSpec((1, OUT_CH), lambda i: (0, 0)),
    ],
    out_specs=pl.BlockSpec((_BM, OUT_CH), lambda i: (i, 0)),
)

_BN = 4096
_GN = -(-N_NODES // _BN)

_adj_call = pl.pallas_call(
    _adj_body,
    grid=(_GM, _GN),
    out_shape=jax.ShapeDtypeStruct((N_NODES, N_NODES), jnp.float32),
    in_specs=[
        pl.BlockSpec((_BM, OUT_CH), lambda i, j: (i, 0)),
        pl.BlockSpec((_BN, OUT_CH), lambda i, j: (j, 0)),
    ],
    out_specs=pl.BlockSpec((_BM, _BN), lambda i, j: (i, j)),
    compiler_params=pltpu.CompilerParams(
        dimension_semantics=("parallel", "parallel")),
)


def kernel(x, edge_index, W1, b1, W2, b2):
    ei = edge_index.astype(jnp.int32)
    src, dst = ei[0], ei[1]

    degp = _deg_call(dst.reshape(NW, EPT_RAW))
    dinv2d = _dinv_call(degp)
    dinv = dinv2d.reshape(-1)[:N_NODES, None]

    padn = STAGE * ACH - N_EDGES
    trash = N_NODES + jnp.arange(padn, dtype=jnp.int32) % (NPAD - N_NODES)
    dstp = jnp.concatenate([dst, trash]).reshape(STAGE, ACH)
    srcp = jnp.concatenate(
        [src, jnp.zeros((padn,), jnp.int32)]).reshape(STAGE, ACH)

    zeros64 = jnp.zeros((RPT, OUT_CH), jnp.float32)
    hs1a, hs1b = _hs1_call(x, W1, dinv)
    aggp1a = _agg64(hs1a, srcp, dstp, zeros64)
    aggp1b = _agg64(hs1b, srcp, dstp, zeros64)
    hs2 = _hs2_call(aggp1a, aggp1a, aggp1b, aggp1b, hs1a, hs1b, dinv,
                    b1.reshape(1, HID_CH), W2)
    aggp2 = _agg64(hs2, srcp, dstp, zeros64)
    z = _z_call(aggp2, aggp2, hs2, dinv, b2.reshape(1, OUT_CH))
    adj = _adj_call(z, z)
    return (z, adj)
